# Initial kernel scaffold; baseline (speedup 1.0000x reference)
import jax
import jax.numpy as jnp
from jax.experimental import pallas as pl


def kernel(Z, Y):
    raise NotImplementedError("write your pallas kernel here")


# speedup vs baseline: 13.0108x; 13.0108x over previous
import functools

import jax
import jax.numpy as jnp
from jax.experimental import pallas as pl
from jax.experimental.pallas import tpu as pltpu

B = 4096
D = 1024
K1 = 1
K2 = 10


def _sim_kernel(y_ref, z_ref, sim_ref, acc_ref, diag_ref, rank_ref, *, t, gj):
    i = pl.program_id(0)
    j = pl.program_id(1)
    c = jax.lax.rem(i + j, gj)

    y = y_ref[...]
    z = z_ref[...]
    ry = jax.lax.rsqrt(jnp.sum(y * y, axis=1, keepdims=True))
    rz = jax.lax.rsqrt(jnp.sum(z * z, axis=1, keepdims=True))
    yn = y * ry
    zn = z * rz
    sim = jax.lax.dot_general(
        yn, zn, (((1,), (1,)), ((), ())),
        preferred_element_type=jnp.float32,
        precision=jax.lax.Precision.HIGHEST,
    )
    sim_ref[...] = sim

    row_l = jax.lax.broadcasted_iota(jnp.int32, (t, t), 0)
    col_l = jax.lax.broadcasted_iota(jnp.int32, (t, t), 1)
    grow = i * t + row_l
    gcol = c * t + col_l

    @pl.when(j == 0)
    def _extract_diag():
        dmask = (row_l == col_l).astype(jnp.float32)
        diag_ref[...] = jnp.sum(sim * dmask, axis=1, keepdims=True)

    d = diag_ref[...]
    beats = (sim > d) | ((sim == d) & (gcol < grow))
    cnt = jnp.sum(beats.astype(jnp.int32), axis=1, keepdims=True)

    @pl.when(j == 0)
    def _init_rank():
        rank_ref[...] = cnt

    @pl.when(j > 0)
    def _acc_rank():
        rank_ref[...] = rank_ref[...] + cnt

    @pl.when(j == gj - 1)
    def _finish_rows():
        rank = rank_ref[...]
        h1 = jnp.sum((rank < K1).astype(jnp.float32))
        h2 = jnp.sum((rank < K2).astype(jnp.float32))
        arow = jax.lax.broadcasted_iota(jnp.int32, (8, 128), 0)
        acol = jax.lax.broadcasted_iota(jnp.int32, (8, 128), 1)
        tile = jnp.where((arow == 0) & (acol == 0), h1, 0.0) + \
               jnp.where((arow == 0) & (acol == 1), h2, 0.0)

        @pl.when(i == 0)
        def _():
            acc_ref[...] = tile

        @pl.when(i > 0)
        def _():
            acc_ref[...] = acc_ref[...] + tile


@functools.partial(jax.jit, static_argnames=("t",))
def _run(Z, Y, t=512):
    g = B // t
    kern = functools.partial(_sim_kernel, t=t, gj=g)
    sim, acc = pl.pallas_call(
        kern,
        grid=(g, g),
        in_specs=[
            pl.BlockSpec((t, D), lambda i, j: (i, 0)),
            pl.BlockSpec((t, D), lambda i, j, _g=g: ((i + j) % _g, 0)),
        ],
        out_specs=[
            pl.BlockSpec((t, t), lambda i, j, _g=g: (i, (i + j) % _g)),
            pl.BlockSpec((8, 128), lambda i, j: (0, 0)),
        ],
        out_shape=[
            jax.ShapeDtypeStruct((B, B), jnp.float32),
            jax.ShapeDtypeStruct((8, 128), jnp.float32),
        ],
        scratch_shapes=[
            pltpu.VMEM((t, 1), jnp.float32),
            pltpu.VMEM((t, 1), jnp.int32),
        ],
    )(Y, Z)
    return acc[0, :2] / B, sim


def kernel(Z, Y):
    accs, sim = _run(Z, Y)
    return accs, sim


# speedup vs baseline: 29.6893x; 2.2819x over previous
import functools

import jax
import jax.numpy as jnp
from jax.experimental import pallas as pl
from jax.experimental.pallas import tpu as pltpu

B = 4096
D = 1024
K1 = 1
K2 = 10


def _sim_kernel(y_ref, z_ref, sim_ref, acc_ref, diag_ref, rank_ref, *, t, gj):
    i = pl.program_id(0)
    j = pl.program_id(1)
    c = jax.lax.rem(i + j, gj)

    y = y_ref[...]
    z = z_ref[...]
    ry = jax.lax.rsqrt(jnp.sum(y * y, axis=1, keepdims=True))
    rz = jax.lax.rsqrt(jnp.sum(z * z, axis=1, keepdims=True))
    yn = (y * ry).astype(jnp.bfloat16)
    zn = (z * rz).astype(jnp.bfloat16)
    sim = jax.lax.dot_general(
        yn, zn, (((1,), (1,)), ((), ())),
        preferred_element_type=jnp.float32,
    )
    sim_ref[...] = sim

    row_l = jax.lax.broadcasted_iota(jnp.int32, (t, t), 0)
    col_l = jax.lax.broadcasted_iota(jnp.int32, (t, t), 1)
    grow = i * t + row_l
    gcol = c * t + col_l

    @pl.when(j == 0)
    def _extract_diag():
        dmask = (row_l == col_l).astype(jnp.float32)
        diag_ref[...] = jnp.sum(sim * dmask, axis=1, keepdims=True)

    d = diag_ref[...]
    beats = (sim > d) | ((sim == d) & (gcol < grow))
    cnt = jnp.sum(beats.astype(jnp.int32), axis=1, keepdims=True)

    @pl.when(j == 0)
    def _init_rank():
        rank_ref[...] = cnt

    @pl.when(j > 0)
    def _acc_rank():
        rank_ref[...] = rank_ref[...] + cnt

    @pl.when(j == gj - 1)
    def _finish_rows():
        rank = rank_ref[...]
        h1 = jnp.sum((rank < K1).astype(jnp.float32))
        h2 = jnp.sum((rank < K2).astype(jnp.float32))
        arow = jax.lax.broadcasted_iota(jnp.int32, (8, 128), 0)
        acol = jax.lax.broadcasted_iota(jnp.int32, (8, 128), 1)
        tile = jnp.where((arow == 0) & (acol == 0), h1, 0.0) + \
               jnp.where((arow == 0) & (acol == 1), h2, 0.0)

        @pl.when(i == 0)
        def _():
            acc_ref[...] = tile

        @pl.when(i > 0)
        def _():
            acc_ref[...] = acc_ref[...] + tile


@functools.partial(jax.jit, static_argnames=("t",))
def _run(Z, Y, t=512):
    g = B // t
    kern = functools.partial(_sim_kernel, t=t, gj=g)
    sim, acc = pl.pallas_call(
        kern,
        grid=(g, g),
        in_specs=[
            pl.BlockSpec((t, D), lambda i, j: (i, 0)),
            pl.BlockSpec((t, D), lambda i, j, _g=g: ((i + j) % _g, 0)),
        ],
        out_specs=[
            pl.BlockSpec((t, t), lambda i, j, _g=g: (i, (i + j) % _g)),
            pl.BlockSpec((8, 128), lambda i, j: (0, 0)),
        ],
        out_shape=[
            jax.ShapeDtypeStruct((B, B), jnp.float32),
            jax.ShapeDtypeStruct((8, 128), jnp.float32),
        ],
        scratch_shapes=[
            pltpu.VMEM((t, 1), jnp.float32),
            pltpu.VMEM((t, 1), jnp.int32),
        ],
    )(Y, Z)
    return acc[0, :2] / B, sim


def kernel(Z, Y):
    accs, sim = _run(Z, Y)
    return accs, sim


# speedup vs baseline: 57.6679x; 1.9424x over previous
import functools

import jax
import jax.numpy as jnp
from jax.experimental import pallas as pl
from jax.experimental.pallas import tpu as pltpu

B = 4096
D = 1024
K1 = 1
K2 = 10


def _chunk_sum(mask, t):
    acc = mask[:, 0:128].astype(jnp.int32)
    for k in range(1, t // 128):
        acc = acc + mask[:, k * 128:(k + 1) * 128].astype(jnp.int32)
    return acc


def _sim_kernel(y_ref, z_ref, sim_ref, acc_ref,
                zn_ref, yn_ref, diag_ref, cnt_ref, *, t, g):
    i = pl.program_id(0)
    j = pl.program_id(1)
    c = jax.lax.rem(i + j, g)

    @pl.when(i == 0)
    def _fill_zn():
        z = z_ref[...]
        rz = jax.lax.rsqrt(jnp.sum(z * z, axis=1, keepdims=True))
        zn_ref[pl.ds(c * t, t), :] = (z * rz).astype(jnp.bfloat16)

    @pl.when(j == 0)
    def _fill_yn():
        y = y_ref[...]
        ry = jax.lax.rsqrt(jnp.sum(y * y, axis=1, keepdims=True))
        yn_ref[...] = (y * ry).astype(jnp.bfloat16)

    yn = yn_ref[...]
    zn = zn_ref[pl.ds(c * t, t), :]
    sim = jax.lax.dot_general(
        yn, zn, (((1,), (1,)), ((), ())),
        preferred_element_type=jnp.float32,
    )
    sim_ref[...] = sim

    @pl.when(j == 0)
    def _diag_tile():
        row_l = jax.lax.broadcasted_iota(jnp.int32, (t, t), 0)
        col_l = jax.lax.broadcasted_iota(jnp.int32, (t, t), 1)
        dmask = (row_l == col_l).astype(jnp.float32)
        dg = jnp.sum(sim * dmask, axis=1, keepdims=True)
        diag_ref[...] = dg
        beats = (sim > dg) | ((sim == dg) & (col_l < row_l))
        cnt_ref[...] = _chunk_sum(beats, t)

    @pl.when(j > 0)
    def _off_tile():
        d = diag_ref[...]

        @pl.when(i + j < g)
        def _():
            cnt_ref[...] = cnt_ref[...] + _chunk_sum(sim > d, t)

        @pl.when(i + j >= g)
        def _():
            cnt_ref[...] = cnt_ref[...] + _chunk_sum(sim >= d, t)

    @pl.when(j == g - 1)
    def _finish_rows():
        rank = jnp.sum(cnt_ref[...], axis=1, keepdims=True)
        h1 = jnp.sum((rank < K1).astype(jnp.float32))
        h2 = jnp.sum((rank < K2).astype(jnp.float32))
        arow = jax.lax.broadcasted_iota(jnp.int32, (8, 128), 0)
        acol = jax.lax.broadcasted_iota(jnp.int32, (8, 128), 1)
        tile = jnp.where((arow == 0) & (acol == 0), h1, 0.0) + \
               jnp.where((arow == 0) & (acol == 1), h2, 0.0)

        @pl.when(i == 0)
        def _():
            acc_ref[...] = tile

        @pl.when(i > 0)
        def _():
            acc_ref[...] = acc_ref[...] + tile


@functools.partial(jax.jit, static_argnames=("t",))
def _run(Z, Y, t=1024):
    g = B // t
    kern = functools.partial(_sim_kernel, t=t, g=g)
    sim, acc = pl.pallas_call(
        kern,
        grid=(g, g),
        in_specs=[
            pl.BlockSpec((t, D), lambda i, j: (i, 0)),
            pl.BlockSpec((t, D), lambda i, j, _g=g:
                         (jnp.where(i == 0, (i + j) % _g, 0), 0)),
        ],
        out_specs=[
            pl.BlockSpec((t, t), lambda i, j, _g=g: (i, (i + j) % _g)),
            pl.BlockSpec((8, 128), lambda i, j: (0, 0)),
        ],
        out_shape=[
            jax.ShapeDtypeStruct((B, B), jnp.float32),
            jax.ShapeDtypeStruct((8, 128), jnp.float32),
        ],
        scratch_shapes=[
            pltpu.VMEM((B, D), jnp.bfloat16),
            pltpu.VMEM((t, D), jnp.bfloat16),
            pltpu.VMEM((t, 1), jnp.float32),
            pltpu.VMEM((t, 128), jnp.int32),
        ],
    )(Y, Z)
    return acc[0, :2] / B, sim


def kernel(Z, Y):
    accs, sim = _run(Z, Y)
    return accs, sim
